# baseline (speedup 1.0000x reference)
import jax
import jax.numpy as jnp
from jax.experimental import pallas as pl
from jax.experimental.pallas import tpu as pltpu

POOL = 7
SCALE = 0.0625
B, C, H, W = 2, 256, 38, 38
N = 1000
G = 40
NEG = jnp.finfo(jnp.float32).min


def _roi_bins(rois):
    b = rois[:, 0].astype(jnp.int32)
    rs_w = jnp.round(rois[:, 1] * SCALE).astype(jnp.int32)
    rs_h = jnp.round(rois[:, 2] * SCALE).astype(jnp.int32)
    re_w = jnp.round(rois[:, 3] * SCALE).astype(jnp.int32)
    re_h = jnp.round(rois[:, 4] * SCALE).astype(jnp.int32)
    roi_w = jnp.maximum(re_w - rs_w + 1, 1).astype(jnp.float32)
    roi_h = jnp.maximum(re_h - rs_h + 1, 1).astype(jnp.float32)
    bin_w = roi_w / POOL
    bin_h = roi_h / POOL
    p = jnp.arange(POOL, dtype=jnp.float32)
    hstart = jnp.clip(jnp.floor(p[None, :] * bin_h[:, None]).astype(jnp.int32) + rs_h[:, None], 0, H)
    hend = jnp.clip(jnp.ceil((p[None, :] + 1.0) * bin_h[:, None]).astype(jnp.int32) + rs_h[:, None], 0, H)
    wstart = jnp.clip(jnp.floor(p[None, :] * bin_w[:, None]).astype(jnp.int32) + rs_w[:, None], 0, W)
    wend = jnp.clip(jnp.ceil((p[None, :] + 1.0) * bin_w[:, None]).astype(jnp.int32) + rs_w[:, None], 0, W)

    len_w = wend - wstart
    kw = (len_w >= 2).astype(jnp.int32) + (len_w >= 4).astype(jnp.int32)
    wA = jnp.clip(wstart, 0, W - 1)
    wB = jnp.clip(wend - (1 << kw), 0, W - 1)
    base = (kw * B + b[:, None]) * W
    rowA = base + wA
    rowB = base + wB

    hbase = (hstart // 8) * 8
    hlo = hstart - hbase
    hhi = jnp.minimum(hend - hbase, 16)

    valid = ((hend - hstart) > 0)[:, :, None] & (len_w > 0)[:, None, :]
    vbits = jnp.sum(valid.astype(jnp.int32) << jnp.arange(POOL)[None, None, :], axis=2)

    packed = jnp.concatenate([rowA, rowB, hbase, hlo, hhi, vbits], axis=1)
    return packed.reshape(N, 1, 6 * POOL).astype(jnp.int32)


def _kernel_body(idx_ref, fmap_ref, out_ref, tab_ref, colmax_ref):
    i = pl.program_id(0)

    @pl.when(i == 0)
    def _build_tables():
        for bb in range(B):
            f = fmap_ref[bb]
            t1 = jnp.maximum(f, jnp.concatenate([f[1:], f[W - 1:]], axis=0))
            t2 = jnp.maximum(t1, jnp.concatenate([t1[2:], t1[W - 2:]], axis=0))
            tab_ref[pl.ds((0 * B + bb) * W, W)] = f
            tab_ref[pl.ds((1 * B + bb) * W, W)] = t1
            tab_ref[pl.ds((2 * B + bb) * W, W)] = t2
        colmax_ref[:, :, 32:48, :] = jnp.full((2, POOL, 16, C), NEG, jnp.float32)

    iota16 = jax.lax.broadcasted_iota(jnp.int32, (1, 16, 1), 1)
    iota7 = jax.lax.broadcasted_iota(jnp.int32, (POOL, 1), 0)

    def one_roi(g, buf):
        for pw in range(POOL):
            rA = idx_ref[g, 0, pw]
            rB = idx_ref[g, 0, POOL + pw]
            colmax_ref[buf, pw, 0:H, :] = jnp.maximum(tab_ref[rA], tab_ref[rB])

        for ph in range(POOL):
            hb = pl.multiple_of(idx_ref[g, 0, 2 * POOL + ph], 8)
            lo = idx_ref[g, 0, 3 * POOL + ph]
            hi = idx_ref[g, 0, 4 * POOL + ph]
            vbits = idx_ref[g, 0, 5 * POOL + ph]
            win = colmax_ref[buf, :, pl.ds(hb, 16), :]
            madd = jnp.where((iota16 >= lo) & (iota16 < hi), 0.0, NEG)
            m = jnp.max(win + madd, axis=1)
            vmask = (jax.lax.shift_right_logical(vbits, iota7) & 1) > 0
            m = jnp.where(vmask, m, 0.0)
            for pw in range(POOL):
                out_ref[ph * POOL + pw, g, :] = m[pw]

    def roi_pair(j, _):
        one_roi(2 * j, 0)
        one_roi(2 * j + 1, 1)
        return ()

    jax.lax.fori_loop(0, G // 2, roi_pair, (), unroll=False)


def kernel(features, rois):
    fmap = jnp.transpose(features, (0, 3, 2, 1))
    packed = _roi_bins(rois)

    out = pl.pallas_call(
        _kernel_body,
        grid=(N // G,),
        in_specs=[
            pl.BlockSpec((G, 1, 6 * POOL), lambda i: (i, 0, 0), memory_space=pltpu.SMEM),
            pl.BlockSpec((B, W, H, C), lambda i: (0, 0, 0, 0)),
        ],
        out_specs=pl.BlockSpec((POOL * POOL, G, C), lambda i: (0, i, 0)),
        out_shape=jax.ShapeDtypeStruct((POOL * POOL, N, C), jnp.float32),
        scratch_shapes=[
            pltpu.VMEM((3 * B * W, H, C), jnp.float32),
            pltpu.VMEM((2, POOL, 48, C), jnp.float32),
        ],
    )(packed, fmap)
    return jnp.transpose(out.reshape(POOL, POOL, N, C), (2, 3, 0, 1))
